# baseline (speedup 1.0000x reference)
import functools

import jax
import jax.numpy as jnp
from jax import lax
from jax.experimental import pallas as pl
from jax.experimental.pallas import tpu as pltpu
from jax.experimental.pallas import tpu_sc as plsc

_L = 16
_GROUP = 256
_BLK = 128
_CHUNK = 1024
_PCAP = 1024
_NSUB = 16
_GSUB = 8
_SUBCAP = 16384
_NRING = 3
_FILLS = 4


def _popcnt(mask):
    return plsc.all_reduce_population_count(mask)[0]


@functools.lru_cache(maxsize=None)
def _build(n_tokens: int, vocab: int, dim: int):
    info = plsc.get_sparse_core_info()
    nw = info.num_cores * info.num_subcores
    n_groups = (vocab + _GROUP - 1) // _GROUP
    n_chunks = n_tokens // _CHUNK
    srows = _FILLS * _L

    mesh = plsc.VectorSubcoreMesh(core_axis_name="c", subcore_axis_name="s")

    @functools.partial(
        pl.kernel,
        mesh=mesh,
        out_type=jax.ShapeDtypeStruct((n_tokens + _L, 2 * dim), jnp.float32),
        scratch_types=[
            pltpu.VMEM((2, _CHUNK), jnp.int32),
            pltpu.VMEM((n_tokens + _L,), jnp.int32),
            pltpu.VMEM((_SUBCAP + _L,), jnp.int32),
            pltpu.VMEM((_NRING, 2, dim, _BLK), jnp.float32),
            pltpu.VMEM((_PCAP + _L,), jnp.int32),
            pltpu.VMEM((2, srows, 2 * dim), jnp.float32),
            pltpu.VMEM((2, srows), jnp.int32),
            pltpu.SemaphoreType.DMA,
            pltpu.SemaphoreType.DMA,
            pltpu.SemaphoreType.DMA,
        ],
        compiler_params=pltpu.CompilerParams(
            use_tc_tiling_on_sc=True, needs_layout_passes=False),
    )
    def emb_kernel(ids_hbm, tab_hbm, tail_hbm, out_hbm, idsbuf, slabl, subl,
                   blocks, pend, stage, scat_idx, sem_i, sem_b, sem_s):
        wid = lax.axis_index("s") * info.num_cores + lax.axis_index("c")
        gs = wid * n_groups // nw
        ge = (wid + 1) * n_groups // nw
        lane = lax.iota(jnp.int32, _L)
        trash = jnp.full((_L,), n_tokens, jnp.int32) + lane

        def blk_copies(g, par):
            c0 = g * _GROUP
            full = [
                pltpu.make_async_copy(
                    tab_hbm.at[:, pl.ds(c0, _BLK)], blocks.at[par, 0], sem_b),
                pltpu.make_async_copy(
                    tab_hbm.at[:, pl.ds(c0 + _BLK, _BLK)], blocks.at[par, 1],
                    sem_b),
            ]
            part = [
                pltpu.make_async_copy(tail_hbm, blocks.at[par, 0], sem_b),
            ]
            return full, part

        def issue_group(g, par):
            full, part = blk_copies(g, par)

            @pl.when(g < n_groups - 1)
            def _():
                for c in full:
                    c.start()

            @pl.when(g == n_groups - 1)
            def _():
                part[0].start()

        def wait_group(g, par):
            full, part = blk_copies(g, par)

            @pl.when(g < n_groups - 1)
            def _():
                for c in full:
                    c.wait()

            @pl.when(g == n_groups - 1)
            def _():
                part[0].wait()

        def wait_scat(slot):
            pltpu.make_async_copy(
                stage.at[slot], out_hbm.at[scat_idx.at[slot]], sem_s).wait()

        def issue_scat(slot):
            pltpu.make_async_copy(
                stage.at[slot], out_hbm.at[scat_idx.at[slot]], sem_s).start()

        pltpu.make_async_copy(
            ids_hbm.at[pl.ds(0, _CHUNK)], idsbuf.at[0], sem_i).start()
        for s in range(2):
            for f in range(_FILLS):
                scat_idx[s, pl.ds(f * _L, _L)] = trash
            issue_scat(s)

        lo = gs * _GROUP
        hi = ge * _GROUP

        def chunk_body(j, cnt):
            @pl.when(j + 1 < n_chunks)
            def _():
                pltpu.make_async_copy(
                    ids_hbm.at[pl.ds((j + 1) * _CHUNK, _CHUNK)],
                    idsbuf.at[(j + 1) % 2], sem_i).start()

            pltpu.make_async_copy(
                ids_hbm.at[pl.ds(j * _CHUNK, _CHUNK)],
                idsbuf.at[j % 2], sem_i).wait()

            def vec_body(k, c):
                v = idsbuf[j % 2, pl.ds(k * _L, _L)]
                pos = j * _CHUNK + k * _L + lane
                m = jnp.logical_and(v >= lo, v < hi)
                packed = (v - lo) * 65536 + pos
                plsc.store_compressed(slabl.at[pl.ds(c, _L)], packed, mask=m)
                return c + _popcnt(m)

            return lax.fori_loop(0, _CHUNK // _L, vec_body, cnt)

        cnt = lax.fori_loop(0, n_chunks, chunk_body, jnp.int32(0))
        n_vecs = (cnt + _L - 1) // _L
        overflow = cnt > _SUBCAP

        def run_partition():
            offs = [jnp.int32(0)]
            for b in range(_NSUB):
                b_lo = b * _GSUB * _GROUP << 16
                b_hi = (b + 1) * _GSUB * _GROUP << 16

                def part_body(k, c, b_lo=b_lo, b_hi=b_hi):
                    v = slabl[pl.ds(k * _L, _L)]
                    valid = (k * _L + lane) < cnt
                    m = jnp.logical_and(v >= b_lo, valid)
                    if b < _NSUB - 1:
                        m = jnp.logical_and(m, v < b_hi)
                    plsc.store_compressed(subl.at[pl.ds(c, _L)], v, mask=m)
                    return c + _popcnt(m)

                offs.append(lax.fori_loop(0, n_vecs, part_body, offs[-1]))
            return tuple(offs)

        offs = lax.cond(
            overflow,
            lambda: tuple(jnp.int32(0) for _ in range(_NSUB + 1)),
            run_partition,
        )

        def drain(pcnt, st, par, grel_base):
            par_s = jnp.full((_L,), par, jnp.int32)

            def d_body(k, st2):
                so, sc = st2
                slot = sc % 2
                slot_s = jnp.full((_L,), slot, jnp.int32)

                @pl.when(so == 0)
                def _():
                    wait_scat(slot)

                pv = pend[pl.ds(k * _L, _L)]
                valid = (k * _L + lane) < pcnt
                ptv = jnp.where(valid, jnp.bitwise_and(pv, 0xFFFF), trash)
                prv = jnp.where(
                    valid, jnp.right_shift(pv, 16) - grel_base, 0)
                jv = jnp.right_shift(prv, 7)
                rlv = jnp.bitwise_and(prv, _BLK - 1)
                scat_idx[slot, pl.ds(so * _L, _L)] = ptv
                row_v = so * _L + lane

                def c_body(ci, _):
                    for u in range(4):
                        c_v = jnp.full((_L,), ci * 4 + u, jnp.int32)
                        vals = plsc.load_gather(
                            blocks, [par_s, jv, c_v, rlv])
                        plsc.store_scatter(
                            stage, [slot_s, row_v, c_v], vals)
                    return 0

                lax.fori_loop(0, dim // 4, c_body, 0)

                return lax.cond(
                    so + 1 == _FILLS,
                    lambda: (issue_scat(slot), (jnp.int32(0), sc + 1))[1],
                    lambda: (so + 1, sc),
                )

            n16 = (pcnt + _L - 1) // _L
            return lax.fori_loop(0, n16, d_body, st)

        def scan_group(g, par, st, list_ref, s_lo, s_hi):
            grel_base = (g - gs) * _GROUP
            p_lo = grel_base * 65536
            p_hi = (grel_base + _GROUP) * 65536

            def scan_body(k, sc):
                pc, st2 = sc
                v = list_ref[pl.ds(s_lo + k * _L, _L)]
                valid = (s_lo + k * _L + lane) < s_hi
                m = jnp.logical_and(
                    jnp.logical_and(v >= p_lo, v < p_hi), valid)
                plsc.store_compressed(pend.at[pl.ds(pc, _L)], v, mask=m)
                pc2 = pc + _popcnt(m)

                return lax.cond(
                    pc2 >= _PCAP - _L,
                    lambda: (jnp.int32(0),
                             drain(pc2, st2, par, grel_base)),
                    lambda: (pc2, st2),
                )

            nv = (s_hi - s_lo + _L - 1) // _L
            pc, st = lax.fori_loop(0, nv, scan_body, (jnp.int32(0), st))
            return drain(pc, st, par, grel_base)

        def finish(st):
            so, sc = st
            slot = sc % 2

            @pl.when(so > 0)
            def _():
                def pad_body(f, _):
                    scat_idx[slot, pl.ds(f * _L, _L)] = trash
                    return 0

                lax.fori_loop(so, _FILLS, pad_body, 0)
                issue_scat(slot)

            wait_scat(0)
            wait_scat(1)

        @pl.when(jnp.logical_not(overflow))
        def _():
            issue_group(gs, 0)

            @pl.when(gs + 1 < ge)
            def _():
                issue_group(gs + 1, 1)

            st = (jnp.int32(0), jnp.int32(0))
            for b in range(_NSUB):
                s_lo, s_hi = offs[b], offs[b + 1]
                gb = gs + b * _GSUB
                nb = jnp.clip(ge - gb, 0, _GSUB)

                def group_body(i, st2, gb=gb, s_lo=s_lo, s_hi=s_hi):
                    g = gb + i
                    par = (g - gs) % _NRING

                    @pl.when(g + 2 < ge)
                    def _():
                        issue_group(g + 2, (g + 2 - gs) % _NRING)

                    wait_group(g, par)
                    return scan_group(g, par, st2, subl, s_lo, s_hi)

                st = lax.fori_loop(0, nb, group_body, st)
            finish(st)

        @pl.when(overflow)
        def _():
            issue_group(gs, 0)

            @pl.when(gs + 1 < ge)
            def _():
                issue_group(gs + 1, 1)

            def group_body(g, st2):
                par = (g - gs) % _NRING

                @pl.when(g + 2 < ge)
                def _():
                    issue_group(g + 2, (g + 2 - gs) % _NRING)

                wait_group(g, par)
                return scan_group(g, par, st2, slabl, jnp.int32(0), cnt)

            st = lax.fori_loop(gs, ge, group_body,
                               (jnp.int32(0), jnp.int32(0)))
            finish(st)

    return emb_kernel


def kernel(input_ids, embedding_table):
    batch, seq = input_ids.shape
    vocab, dim = embedding_table.shape
    n_tokens = batch * seq
    tab_t = jnp.transpose(embedding_table)
    last_cols = vocab - (vocab // _BLK) * _BLK
    tail = jnp.pad(tab_t[:, vocab - last_cols:],
                   ((0, 0), (0, _BLK - last_cols)))
    ids_flat = input_ids.reshape(n_tokens).astype(jnp.int32)
    out = _build(n_tokens, vocab, dim)(ids_flat, tab_t, tail)
    return out[:n_tokens, :dim].reshape(batch, seq, dim)
